# baseline (speedup 1.0000x reference)
import functools

import jax
import jax.numpy as jnp
from jax import lax
from jax.experimental import pallas as pl
from jax.experimental.pallas import tpu as pltpu
from jax.experimental.pallas import tpu_sc as plsc

_WIN = 72
_OFF = 78 - _WIN


def _sc_center_tap(f_flat, w_flat):
    mesh = plsc.VectorSubcoreMesh(
        core_axis_name="c", subcore_axis_name="s", num_cores=1, num_subcores=1
    )

    @functools.partial(
        pl.kernel,
        mesh=mesh,
        out_type=jax.ShapeDtypeStruct((1, 3), jnp.float32),
        scratch_types=[
            pltpu.VMEM((16,), jnp.float32),
            pltpu.VMEM((32,), jnp.float32),
            pltpu.VMEM((16,), jnp.float32),
            pltpu.SemaphoreType.DMA,
            pltpu.SemaphoreType.DMA,
        ],
        compiler_params=pltpu.CompilerParams(skip_device_barrier=True),
    )
    def body(f_hbm, w_hbm, out_hbm, f_v, w_v, o_v, sem_f, sem_w):
        cp_f = pltpu.async_copy(f_hbm, f_v.at[pl.ds(0, 2)], sem_f)
        cp_w = pltpu.async_copy(
            w_hbm.at[pl.ds(_WIN, 16)], w_v.at[pl.ds(0, 16)], sem_w
        )
        cp_f.wait()
        cp_w.wait()
        fv = f_v[...]
        w0 = w_v[pl.ds(_OFF, 16)]
        w1 = w_v[pl.ds(_OFF + 3, 16)]
        f0 = jnp.full((16,), fv[0], jnp.float32)
        f1 = jnp.full((16,), fv[1], jnp.float32)
        o_v[...] = w0 * f0 + w1 * f1
        pltpu.sync_copy(o_v.at[pl.ds(0, 3)], out_hbm.at[0])

    return body(f_flat, w_flat)


def kernel(features, indices, W):
    del indices
    return _sc_center_tap(features.reshape(2), W.reshape(162))


# speedup vs baseline: 1.0036x; 1.0036x over previous
import functools

import jax
import jax.numpy as jnp
from jax import lax
from jax.experimental import pallas as pl
from jax.experimental.pallas import tpu as pltpu
from jax.experimental.pallas import tpu_sc as plsc

_WIN = 72
_OFF = 78 - _WIN


def _sc_center_tap(f_flat, w_flat):
    mesh = plsc.VectorSubcoreMesh(
        core_axis_name="c", subcore_axis_name="s", num_cores=1, num_subcores=1
    )

    @functools.partial(
        pl.kernel,
        mesh=mesh,
        out_type=jax.ShapeDtypeStruct((1, 3), jnp.float32),
        scratch_types=[
            pltpu.VMEM((16,), jnp.float32),
            pltpu.VMEM((32,), jnp.float32),
            pltpu.VMEM((16,), jnp.float32),
            pltpu.SemaphoreType.DMA,
            pltpu.SemaphoreType.DMA,
        ],
        compiler_params=pltpu.CompilerParams(
            disable_bounds_checks=True, disable_semaphore_checks=True
        ),
    )
    def body(f_hbm, w_hbm, out_hbm, f_v, w_v, o_v, sem_f, sem_w):
        cp_f = pltpu.async_copy(f_hbm, f_v.at[pl.ds(0, 2)], sem_f)
        cp_w = pltpu.async_copy(
            w_hbm.at[pl.ds(_WIN, 16)], w_v.at[pl.ds(0, 16)], sem_w
        )
        cp_f.wait()
        cp_w.wait()
        fv = f_v[...]
        w0 = w_v[pl.ds(_OFF, 16)]
        w1 = w_v[pl.ds(_OFF + 3, 16)]
        f0 = jnp.full((16,), fv[0], jnp.float32)
        f1 = jnp.full((16,), fv[1], jnp.float32)
        o_v[...] = w0 * f0 + w1 * f1
        pltpu.sync_copy(o_v.at[pl.ds(0, 3)], out_hbm.at[0])

    return body(f_flat, w_flat)


def kernel(features, indices, W):
    del indices
    return _sc_center_tap(features.reshape(2), W.reshape(162))
